# baseline (speedup 1.0000x reference)
import jax
import jax.numpy as jnp
from jax.experimental import pallas as pl
from jax.experimental.pallas import tpu as pltpu


def _body(x_ref, a_ref, b_ref):
    v = x_ref[0]
    a_ref[0] = v
    b_ref[0] = v


def kernel(x):
    a, b = pl.pallas_call(
        _body,
        grid=(1,),
        in_specs=[pl.BlockSpec((128,), lambda i: (0,), memory_space=pltpu.SMEM)],
        out_specs=(pl.BlockSpec(memory_space=pltpu.SMEM),
                   pl.BlockSpec(memory_space=pltpu.SMEM)),
        out_shape=(jax.ShapeDtypeStruct((1,), jnp.float32),
                   jax.ShapeDtypeStruct((1,), jnp.float32)),
    )(x)
    return (a.reshape(()), b.reshape(()))
